# Initial kernel scaffold; baseline (speedup 1.0000x reference)
import jax
import jax.numpy as jnp
from jax.experimental import pallas as pl


def kernel(params, perm_index):
    raise NotImplementedError("write your pallas kernel here")


# speedup vs baseline: 3.9818x; 3.9818x over previous
import jax
import jax.numpy as jnp
from jax.experimental import pallas as pl
from jax.experimental.pallas import tpu as pltpu


_RB = 128


def _copy_body(perm_ref, params_ref, out_ref):
    i = pl.program_id(0)
    rb = pl.program_id(1)
    for j in range(8):
        f = perm_ref[i * 8 + j]
        out_ref[0, :, j, :] = params_ref[f, pl.ds(rb * _RB, _RB), :]


def kernel(params, perm_index):
    return pl.pallas_call(
        _copy_body,
        grid_spec=pltpu.PrefetchScalarGridSpec(
            num_scalar_prefetch=1,
            grid=(8, 512 // _RB),
            in_specs=[
                pl.BlockSpec((10, 512, 1024), lambda i, rb, perm: (0, 0, 0)),
            ],
            out_specs=pl.BlockSpec(
                (1, _RB, 8, 1024), lambda i, rb, perm: (i, rb, 0, 0)
            ),
        ),
        out_shape=jax.ShapeDtypeStruct((8, 512, 8, 1024), jnp.float32),
    )(perm_index.reshape(64), params)
